# baseline (speedup 1.0000x reference)
import jax
import jax.numpy as jnp
from jax.experimental import pallas as pl
from jax.experimental.pallas import tpu as pltpu

DIM = 512
HID = 2048
E = 8
NBUF = 3


def _gate_weights(logits):
    T = logits.shape[0]
    col = jax.lax.broadcasted_iota(jnp.int32, (T, E), 1)
    m1 = jnp.max(logits, axis=1, keepdims=True)
    big = jnp.int32(E)
    idx1 = jnp.min(jnp.where(logits == m1, col, big), axis=1, keepdims=True)
    masked = jnp.where(col == idx1, -jnp.inf, logits)
    m2 = jnp.max(masked, axis=1, keepdims=True)
    idx2 = jnp.min(jnp.where(masked == m2, col, big), axis=1, keepdims=True)
    e2 = jnp.exp(m2 - m1)
    p1 = 1.0 / (1.0 + e2)
    p2 = 1.0 - p1
    return jnp.where(col == idx1, p1, jnp.where(col == idx2, p2, 0.0))


def _moe_body(x_ref, gw_ref, w1_hbm, w2_hbm, o_ref, w1buf, w2buf, sems):
    def copy1(e, b):
        return pltpu.make_async_copy(w1_hbm.at[e], w1buf.at[b], sems.at[b, 0])

    def copy2(e, b):
        return pltpu.make_async_copy(w2_hbm.at[e], w2buf.at[b], sems.at[b, 1])

    for e in range(NBUF):
        copy1(e, e).start()
        copy2(e, e).start()

    xb = x_ref[...]
    logits = jax.lax.dot_general(
        xb, gw_ref[...], (((1,), (1,)), ((), ())),
        preferred_element_type=jnp.float32)
    wf = _gate_weights(logits)

    for e in range(E):
        b = e % NBUF
        copy1(e, b).wait()
        if e < E - 1:
            copy2(e, b).wait()
        hh = jax.lax.dot_general(
            xb, w1buf[b], (((1,), (1,)), ((), ())),
            preferred_element_type=jnp.float32)
        hh = jnp.maximum(hh, 0.0)
        if e == E - 1:
            copy2(e, b).wait()
        y = jax.lax.dot_general(
            hh, w2buf[b], (((1,), (1,)), ((), ())),
            preferred_element_type=jnp.float32)
        contrib = wf[:, e:e + 1] * y
        if e == 0:
            o_ref[...] = contrib
        else:
            o_ref[...] += contrib
        if e + NBUF < E:
            copy1(e + NBUF, b).start()
            copy2(e + NBUF, b).start()


@jax.jit
def kernel(x, gate_w, w1, w2):
    B, N, D = x.shape
    T = B * N
    out = pl.pallas_call(
        _moe_body,
        in_specs=[
            pl.BlockSpec(memory_space=pltpu.VMEM),
            pl.BlockSpec(memory_space=pltpu.VMEM),
            pl.BlockSpec(memory_space=pl.ANY),
            pl.BlockSpec(memory_space=pl.ANY),
        ],
        out_specs=pl.BlockSpec(memory_space=pltpu.VMEM),
        out_shape=jax.ShapeDtypeStruct((T, D), jnp.float32),
        scratch_shapes=[
            pltpu.VMEM((NBUF, HID, DIM), jnp.float32),
            pltpu.VMEM((NBUF, DIM, HID), jnp.float32),
            pltpu.SemaphoreType.DMA((NBUF, 2)),
        ],
    )(x.reshape(T, D), gate_w, w1, w2)
    return out.reshape(B, N, D)
